# baseline (speedup 1.0000x reference)
import functools

import jax
import jax.numpy as jnp
from jax import lax
from jax.experimental import pallas as pl
from jax.experimental.pallas import tpu as pltpu
from jax.experimental.pallas import tpu_sc as plsc

_IN_DIM = 3
_OUT_DIM = 32
_PAD_DIM = 128
_BIN_NUM = 65536
_N_POINTS = 262144

_NC = 2
_NS = 16
_NW = _NC * _NS
_PPW = _N_POINTS // _NW
_CHUNK = 128
_NCHUNK = _PPW // _CHUNK
_VPA = _CHUNK // 16


def _body(x0_hbm, x1_hbm, x2_hbm, table_hbm, out_hbm,
          x0_v, x1_v, x2_v, idx_v, r0, r1, r2, o_v, sem):
    wid = lax.axis_index("s") * _NC + lax.axis_index("c")
    wbase = wid * _PPW

    xs = (x0_v, x1_v, x2_v)
    for a, xh in enumerate((x0_hbm, x1_hbm, x2_hbm)):
        pltpu.sync_copy(xh.at[pl.ds(wbase, _PPW)], xs[a])

    rows = (r0, r1, r2)

    def chunk_body(ci, carry):
        cbase = ci * _CHUNK

        for a in range(_IN_DIM):
            for v in range(_VPA):
                xv = xs[a][pl.ds(cbase + v * 16, 16)]
                idf = (xv + 1.0) * (0.5 * (_BIN_NUM - 1))
                ii = idf.astype(jnp.int32)
                ii = jnp.maximum(jnp.minimum(ii, _BIN_NUM - 1), 0)
                ii = ii + a * _BIN_NUM
                idx_v[a, pl.ds(v * 16, 16)] = ii

        cps = []
        for a in range(_IN_DIM):
            cps.append(
                pltpu.async_copy(
                    table_hbm.at[idx_v.at[a]],
                    rows[a],
                    sem,
                )
            )
        for cp in cps:
            cp.wait()

        def mean_body(p, c2):
            for u in range(4):
                for h in range(2):
                    s = pl.ds(h * 16, 16)
                    q = p * 4 + u
                    acc = r0[q, s] + r1[q, s] + r2[q, s]
                    o_v[q, s] = acc * (1.0 / 3.0)
            return c2

        lax.fori_loop(0, _CHUNK // 4, mean_body, 0, unroll=False)

        pltpu.sync_copy(o_v, out_hbm.at[pl.ds(wbase + cbase, _CHUNK)])
        return carry

    lax.fori_loop(0, _NCHUNK, chunk_body, 0, unroll=False)


@jax.jit
def _run(x0, x1, x2, table_padded):
    mesh = plsc.VectorSubcoreMesh(core_axis_name="c", subcore_axis_name="s")
    f = pl.kernel(
        _body,
        out_type=jax.ShapeDtypeStruct((_N_POINTS, _OUT_DIM), jnp.float32),
        mesh=mesh,
        scratch_types=[
            pltpu.VMEM((_PPW,), jnp.float32),
            pltpu.VMEM((_PPW,), jnp.float32),
            pltpu.VMEM((_PPW,), jnp.float32),
            pltpu.VMEM((_IN_DIM, _CHUNK), jnp.int32),
            pltpu.VMEM((_CHUNK, _PAD_DIM), jnp.float32),
            pltpu.VMEM((_CHUNK, _PAD_DIM), jnp.float32),
            pltpu.VMEM((_CHUNK, _PAD_DIM), jnp.float32),
            pltpu.VMEM((_CHUNK, _OUT_DIM), jnp.float32),
            pltpu.SemaphoreType.DMA,
        ],
    )
    return f(x0, x1, x2, table_padded)


def kernel(in_tensor, table):
    x_t = in_tensor.T
    table_padded = jnp.pad(table, ((0, 0), (0, _PAD_DIM - _OUT_DIM)))
    return _run(x_t[0], x_t[1], x_t[2], table_padded)


# speedup vs baseline: 3.6295x; 3.6295x over previous
import functools

import jax
import jax.numpy as jnp
from jax import lax
from jax.experimental import pallas as pl
from jax.experimental.pallas import tpu as pltpu
from jax.experimental.pallas import tpu_sc as plsc

_IN_DIM = 3
_OUT_DIM = 32
_BIN_NUM = 65536
_N_POINTS = 262144

_NC = 2
_NS = 16
_NW = _NC * _NS
_PPW = _N_POINTS // _NW
_CHUNK = 512
_NCHUNK = _PPW // _CHUNK
_VPA = _CHUNK // 16
_BURSTS = (_IN_DIM * _CHUNK) // 128
_BPA = _CHUNK // 128


def _body(x0_hbm, x1_hbm, x2_hbm, table_hbm, out_hbm,
          x0_v, x1_v, x2_v, idx_v, r0, r1, r2, o_v, sem):
    wid = lax.axis_index("s") * _NC + lax.axis_index("c")
    wbase = wid * _PPW

    xs = (x0_v, x1_v, x2_v)
    for a, xh in enumerate((x0_hbm, x1_hbm, x2_hbm)):
        pltpu.sync_copy(xh.at[pl.ds(wbase, _PPW)], xs[a])

    rows = (r0, r1, r2)
    third = jnp.float32(1.0 / 3.0)

    def chunk_body(ci, carry):
        cbase = ci * _CHUNK

        for a in range(_IN_DIM):
            for v in range(_VPA):
                xv = xs[a][pl.ds(cbase + v * 16, 16)]
                idf = (xv + 1.0) * (0.5 * (_BIN_NUM - 1))
                ii = idf.astype(jnp.int32)
                ii = jnp.maximum(jnp.minimum(ii, _BIN_NUM - 1), 0)
                ii = ii + a * _BIN_NUM
                flat = a * _CHUNK + v * 16
                idx_v[flat // 128, pl.ds(flat % 128, 16)] = ii

        cps = []
        for a in range(_IN_DIM):
            for b in range(_BPA):
                cps.append(
                    pltpu.async_copy(
                        table_hbm.at[idx_v.at[a * _BPA + b]],
                        rows[a].at[pl.ds(b * 128, 128)],
                        sem,
                    )
                )
        for cp in cps:
            cp.wait()

        def mean_body(p, c2):
            for u in range(4):
                q = p * 4 + u
                a0, b0 = plsc.unpack(r0[q], format=plsc.PackFormat.INTERLEAVED)
                a1, b1 = plsc.unpack(r1[q], format=plsc.PackFormat.INTERLEAVED)
                a2, b2 = plsc.unpack(r2[q], format=plsc.PackFormat.INTERLEAVED)
                sa = (a0 + a1 + a2) * third
                sb = (b0 + b1 + b2) * third
                o_v[q] = plsc.pack(sa, sb, format=plsc.PackFormat.INTERLEAVED)
            return c2

        lax.fori_loop(0, _CHUNK // 4, mean_body, 0, unroll=False)

        pltpu.sync_copy(o_v, out_hbm.at[pl.ds(wbase + cbase, _CHUNK)])
        return carry

    lax.fori_loop(0, _NCHUNK, chunk_body, 0, unroll=False)


@jax.jit
def _run(x0, x1, x2, table_bf):
    mesh = plsc.VectorSubcoreMesh(core_axis_name="c", subcore_axis_name="s")
    f = pl.kernel(
        _body,
        out_type=jax.ShapeDtypeStruct((_N_POINTS, _OUT_DIM), jnp.bfloat16),
        mesh=mesh,
        scratch_types=[
            pltpu.VMEM((_PPW,), jnp.float32),
            pltpu.VMEM((_PPW,), jnp.float32),
            pltpu.VMEM((_PPW,), jnp.float32),
            pltpu.VMEM((_BURSTS, 128), jnp.int32),
            pltpu.VMEM((_CHUNK, _OUT_DIM), jnp.bfloat16),
            pltpu.VMEM((_CHUNK, _OUT_DIM), jnp.bfloat16),
            pltpu.VMEM((_CHUNK, _OUT_DIM), jnp.bfloat16),
            pltpu.VMEM((_CHUNK, _OUT_DIM), jnp.bfloat16),
            pltpu.SemaphoreType.DMA,
        ],
        compiler_params=pltpu.CompilerParams(
            use_tc_tiling_on_sc=False, needs_layout_passes=False
        ),
    )
    return f(x0, x1, x2, table_bf)


def kernel(in_tensor, table):
    x_t = in_tensor.T
    table_bf = table.astype(jnp.bfloat16)
    out_bf = _run(x_t[0], x_t[1], x_t[2], table_bf)
    return out_bf.astype(jnp.float32)


# speedup vs baseline: 3.6392x; 1.0027x over previous
import functools

import jax
import jax.numpy as jnp
from jax import lax
from jax.experimental import pallas as pl
from jax.experimental.pallas import tpu as pltpu
from jax.experimental.pallas import tpu_sc as plsc

_IN_DIM = 3
_OUT_DIM = 32
_BIN_NUM = 65536
_N_POINTS = 262144

_NC = 2
_NS = 16
_NW = _NC * _NS
_PPW = _N_POINTS // _NW
_CHUNK = 512
_NCHUNK = _PPW // _CHUNK
_VPA = _CHUNK // 16
_BURSTS = (_IN_DIM * _CHUNK) // 128
_BPA = _CHUNK // 128


def _body(x0_hbm, x1_hbm, x2_hbm, table_hbm, out_hbm,
          x0_v, x1_v, x2_v, idx_v, r0, r1, r2, o_v, sem):
    wid = lax.axis_index("s") * _NC + lax.axis_index("c")
    wbase = wid * _PPW

    xs = (x0_v, x1_v, x2_v)
    for a, xh in enumerate((x0_hbm, x1_hbm, x2_hbm)):
        pltpu.sync_copy(xh.at[pl.ds(wbase, _PPW)], xs[a])

    rows = (r0, r1, r2)
    third = jnp.bfloat16(1.0 / 3.0)

    def chunk_body(ci, carry):
        cbase = ci * _CHUNK

        for a in range(_IN_DIM):
            for v in range(_VPA):
                xv = xs[a][pl.ds(cbase + v * 16, 16)]
                idf = (xv + 1.0) * (0.5 * (_BIN_NUM - 1))
                ii = idf.astype(jnp.int32)
                ii = jnp.maximum(jnp.minimum(ii, _BIN_NUM - 1), 0)
                ii = ii + a * _BIN_NUM
                flat = a * _CHUNK + v * 16
                idx_v[flat // 128, pl.ds(flat % 128, 16)] = ii

        cps = []
        for a in range(_IN_DIM):
            for b in range(_BPA):
                cps.append(
                    pltpu.async_copy(
                        table_hbm.at[idx_v.at[a * _BPA + b]],
                        rows[a].at[pl.ds(b * 128, 128)],
                        sem,
                    )
                )
        for cp in cps:
            cp.wait()

        def mean_body(p, c2):
            for u in range(4):
                q = p * 4 + u
                acc = (r0[q] + r1[q] + r2[q]) * third
                o_v[q] = acc
            return c2

        lax.fori_loop(0, _CHUNK // 4, mean_body, 0, unroll=False)

        pltpu.sync_copy(o_v, out_hbm.at[pl.ds(wbase + cbase, _CHUNK)])
        return carry

    lax.fori_loop(0, _NCHUNK, chunk_body, 0, unroll=False)


@jax.jit
def _run(x0, x1, x2, table_bf):
    mesh = plsc.VectorSubcoreMesh(core_axis_name="c", subcore_axis_name="s")
    f = pl.kernel(
        _body,
        out_type=jax.ShapeDtypeStruct((_N_POINTS, _OUT_DIM), jnp.bfloat16),
        mesh=mesh,
        scratch_types=[
            pltpu.VMEM((_PPW,), jnp.float32),
            pltpu.VMEM((_PPW,), jnp.float32),
            pltpu.VMEM((_PPW,), jnp.float32),
            pltpu.VMEM((_BURSTS, 128), jnp.int32),
            pltpu.VMEM((_CHUNK, _OUT_DIM), jnp.bfloat16),
            pltpu.VMEM((_CHUNK, _OUT_DIM), jnp.bfloat16),
            pltpu.VMEM((_CHUNK, _OUT_DIM), jnp.bfloat16),
            pltpu.VMEM((_CHUNK, _OUT_DIM), jnp.bfloat16),
            pltpu.SemaphoreType.DMA,
        ],
        compiler_params=pltpu.CompilerParams(use_tc_tiling_on_sc=False),
    )
    return f(x0, x1, x2, table_bf)


def kernel(in_tensor, table):
    x_t = in_tensor.T
    table_bf = table.astype(jnp.bfloat16)
    out_bf = _run(x_t[0], x_t[1], x_t[2], table_bf)
    return out_bf.astype(jnp.float32)


# speedup vs baseline: 3.7832x; 1.0396x over previous
import functools

import numpy as np
import jax
import jax.numpy as jnp
from jax import lax
from jax.experimental import pallas as pl
from jax.experimental.pallas import tpu as pltpu
from jax.experimental.pallas import tpu_sc as plsc

_IN_DIM = 3
_OUT_DIM = 32
_BIN_NUM = 65536
_N_POINTS = 262144

_NC = 2
_NS = 16
_NW = _NC * _NS
_PPW = _N_POINTS // _NW
_CHUNK = 512
_NCHUNK = _PPW // _CHUNK
_VPA = _CHUNK // 16
_BURSTS = (_IN_DIM * _CHUNK) // 128
_BPA = _CHUNK // 128


def _body(x0_hbm, x1_hbm, x2_hbm, table_hbm, out_hbm,
          x0_v, x1_v, x2_v, idx_v, r0, r1, r2, o_v, sem):
    wid = lax.axis_index("s") * _NC + lax.axis_index("c")
    wbase = wid * _PPW

    xs = (x0_v, x1_v, x2_v)
    for a, xh in enumerate((x0_hbm, x1_hbm, x2_hbm)):
        pltpu.sync_copy(xh.at[pl.ds(wbase, _PPW)], xs[a])

    rows = (r0, r1, r2)
    third = jnp.float32(1.0 / 3.0)

    def chunk_body(ci, carry):
        cbase = ci * _CHUNK

        for a in range(_IN_DIM):
            for v in range(_VPA):
                xv = xs[a][pl.ds(cbase + v * 16, 16)]
                idf = (xv + 1.0) * (0.5 * (_BIN_NUM - 1))
                ii = idf.astype(jnp.int32)
                ii = jnp.maximum(jnp.minimum(ii, _BIN_NUM - 1), 0)
                ii = ii + a * _BIN_NUM
                flat = a * _CHUNK + v * 16
                idx_v[flat // 128, pl.ds(flat % 128, 16)] = ii

        cps = []
        for a in range(_IN_DIM):
            for b in range(_BPA):
                cps.append(
                    pltpu.async_copy(
                        table_hbm.at[idx_v.at[a * _BPA + b]],
                        rows[a].at[pl.ds(b * 128, 128)],
                        sem,
                    )
                )
        for cp in cps:
            cp.wait()

        def mean_body(p, c2):
            for u in range(4):
                q = p * 4 + u
                a0, b0 = plsc.unpack(r0[q], format=plsc.PackFormat.INTERLEAVED)
                a1, b1 = plsc.unpack(r1[q], format=plsc.PackFormat.INTERLEAVED)
                a2, b2 = plsc.unpack(r2[q], format=plsc.PackFormat.INTERLEAVED)
                sa = (a0 + a1 + a2) * third
                sb = (b0 + b1 + b2) * third
                o_v[q, pl.ds(0, 16)] = sa
                o_v[q, pl.ds(16, 16)] = sb
            return c2

        lax.fori_loop(0, _CHUNK // 4, mean_body, 0, unroll=False)

        pltpu.sync_copy(o_v, out_hbm.at[pl.ds(wbase + cbase, _CHUNK)])
        return carry

    lax.fori_loop(0, _NCHUNK, chunk_body, 0, unroll=False)


@jax.jit
def _run(x0, x1, x2, table_bf):
    mesh = plsc.VectorSubcoreMesh(core_axis_name="c", subcore_axis_name="s")
    f = pl.kernel(
        _body,
        out_type=jax.ShapeDtypeStruct((_N_POINTS, _OUT_DIM), jnp.float32),
        mesh=mesh,
        scratch_types=[
            pltpu.VMEM((_PPW,), jnp.float32),
            pltpu.VMEM((_PPW,), jnp.float32),
            pltpu.VMEM((_PPW,), jnp.float32),
            pltpu.VMEM((_BURSTS, 128), jnp.int32),
            pltpu.VMEM((_CHUNK, _OUT_DIM), jnp.bfloat16),
            pltpu.VMEM((_CHUNK, _OUT_DIM), jnp.bfloat16),
            pltpu.VMEM((_CHUNK, _OUT_DIM), jnp.bfloat16),
            pltpu.VMEM((_CHUNK, _OUT_DIM), jnp.float32),
            pltpu.SemaphoreType.DMA,
        ],
        compiler_params=pltpu.CompilerParams(
            use_tc_tiling_on_sc=False, needs_layout_passes=False
        ),
    )
    return f(x0, x1, x2, table_bf)


_COL_PERM = np.empty(_OUT_DIM, dtype=np.int32)
_COL_PERM[0::2] = np.arange(_OUT_DIM // 2)
_COL_PERM[1::2] = np.arange(_OUT_DIM // 2) + _OUT_DIM // 2


def kernel(in_tensor, table):
    x_t = in_tensor.T
    table_bf = table.astype(jnp.bfloat16)[:, _COL_PERM]
    return _run(x_t[0], x_t[1], x_t[2], table_bf)


# speedup vs baseline: 3.7875x; 1.0011x over previous
import functools

import numpy as np
import jax
import jax.numpy as jnp
from jax import lax
from jax.experimental import pallas as pl
from jax.experimental.pallas import tpu as pltpu
from jax.experimental.pallas import tpu_sc as plsc

_IN_DIM = 3
_OUT_DIM = 32
_BIN_NUM = 65536
_N_POINTS = 262144

_NC = 2
_NS = 16
_NW = _NC * _NS
_PPW = _N_POINTS // _NW
_CHUNK = 512
_NCHUNK = _PPW // _CHUNK
_VPA = _CHUNK // 16
_BURSTS = (_IN_DIM * _CHUNK) // 128
_BPA = _CHUNK // 128


def _body(x0_hbm, x1_hbm, x2_hbm, table_hbm, out_hbm,
          x0_v, x1_v, x2_v, idx_v, r0, r1, r2, o_v, o_sh, sem):
    wid = lax.axis_index("s") * _NC + lax.axis_index("c")
    wbase = wid * _PPW

    xs = (x0_v, x1_v, x2_v)
    for a, xh in enumerate((x0_hbm, x1_hbm, x2_hbm)):
        pltpu.sync_copy(xh.at[pl.ds(wbase, _PPW)], xs[a])

    rows = (r0, r1, r2)
    third = jnp.float32(1.0 / 3.0)

    def chunk_body(ci, carry):
        cbase = ci * _CHUNK

        for a in range(_IN_DIM):
            for v in range(_VPA):
                xv = xs[a][pl.ds(cbase + v * 16, 16)]
                idf = (xv + 1.0) * (0.5 * (_BIN_NUM - 1))
                ii = idf.astype(jnp.int32)
                ii = jnp.maximum(jnp.minimum(ii, _BIN_NUM - 1), 0)
                ii = ii + a * _BIN_NUM
                flat = a * _CHUNK + v * 16
                idx_v[flat // 128, pl.ds(flat % 128, 16)] = ii

        cps = []
        for a in range(_IN_DIM):
            for b in range(_BPA):
                cps.append(
                    pltpu.async_copy(
                        table_hbm.at[idx_v.at[a * _BPA + b]],
                        rows[a].at[pl.ds(b * 128, 128)],
                        sem,
                    )
                )
        for cp in cps:
            cp.wait()

        def mean_body(p, c2):
            for u in range(4):
                q = p * 4 + u
                a0, b0 = plsc.unpack(r0[q], format=plsc.PackFormat.INTERLEAVED)
                a1, b1 = plsc.unpack(r1[q], format=plsc.PackFormat.INTERLEAVED)
                a2, b2 = plsc.unpack(r2[q], format=plsc.PackFormat.INTERLEAVED)
                sa = (a0 + a1 + a2) * third
                sb = (b0 + b1 + b2) * third
                o_v[q, pl.ds(0, 16)] = sa
                o_v[q, pl.ds(16, 16)] = sb
            return c2

        lax.fori_loop(0, _CHUNK // 4, mean_body, 0, unroll=False)

        pltpu.sync_copy(o_v, o_sh.at[pl.ds(wid * _CHUNK, _CHUNK)])
        pltpu.sync_copy(o_sh.at[pl.ds(wid * _CHUNK, _CHUNK)],
                        out_hbm.at[pl.ds(wbase + cbase, _CHUNK)])
        return carry

    lax.fori_loop(0, _NCHUNK, chunk_body, 0, unroll=False)


@jax.jit
def _run(x0, x1, x2, table_bf):
    mesh = plsc.VectorSubcoreMesh(core_axis_name="c", subcore_axis_name="s")
    f = pl.kernel(
        _body,
        out_type=jax.ShapeDtypeStruct((_N_POINTS, _OUT_DIM), jnp.float32),
        mesh=mesh,
        scratch_types=[
            pltpu.VMEM((_PPW,), jnp.float32),
            pltpu.VMEM((_PPW,), jnp.float32),
            pltpu.VMEM((_PPW,), jnp.float32),
            pltpu.VMEM((_BURSTS, 128), jnp.int32),
            pltpu.VMEM((_CHUNK, _OUT_DIM), jnp.bfloat16),
            pltpu.VMEM((_CHUNK, _OUT_DIM), jnp.bfloat16),
            pltpu.VMEM((_CHUNK, _OUT_DIM), jnp.bfloat16),
            pltpu.VMEM((_CHUNK, _OUT_DIM), jnp.float32),
            pltpu.VMEM_SHARED((_NW * _CHUNK, _OUT_DIM), jnp.float32),
            pltpu.SemaphoreType.DMA,
        ],
        compiler_params=pltpu.CompilerParams(
            use_tc_tiling_on_sc=False, needs_layout_passes=False
        ),
    )
    return f(x0, x1, x2, table_bf)


_COL_PERM = np.empty(_OUT_DIM, dtype=np.int32)
_COL_PERM[0::2] = np.arange(_OUT_DIM // 2)
_COL_PERM[1::2] = np.arange(_OUT_DIM // 2) + _OUT_DIM // 2


def kernel(in_tensor, table):
    x_t = in_tensor.T
    table_bf = table.astype(jnp.bfloat16)[:, _COL_PERM]
    return _run(x_t[0], x_t[1], x_t[2], table_bf)
